# baseline (speedup 1.0000x reference)
import functools

import jax
import jax.numpy as jnp
from jax import lax
from jax.experimental import pallas as pl
from jax.experimental.pallas import tpu as pltpu
from jax.experimental.pallas import tpu_sc as plsc

NUM_CORES = 2
NUM_SUBCORES = 16
NW = NUM_CORES * NUM_SUBCORES
CHUNK = 512
D = 64


NBUF = 2


@functools.lru_cache(maxsize=None)
def _build(B):
    n_per_w = B // NW
    n_chunks = n_per_w // CHUNK
    assert n_chunks % NBUF == 0
    mesh = plsc.VectorSubcoreMesh(core_axis_name="c", subcore_axis_name="s")

    @functools.partial(
        pl.kernel,
        mesh=mesh,
        out_type=jax.ShapeDtypeStruct((B, D), jnp.float32),
        compiler_params=pltpu.CompilerParams(use_tc_tiling_on_sc=False),
        scratch_types=[
            pltpu.VMEM((n_chunks, CHUNK), jnp.int32),
        ]
        + [pltpu.VMEM((CHUNK, D), jnp.float32) for _ in range(NBUF)]
        + [pltpu.SemaphoreType.DMA for _ in range(2 * NBUF)],
    )
    def emb(table_hbm, idx_hbm, out_hbm, idx_v, *bufs_sems):
        rows = bufs_sems[:NBUF]
        gsem = bufs_sems[NBUF : 2 * NBUF]
        ssem = bufs_sems[2 * NBUF :]
        wid = lax.axis_index("s") * NUM_CORES + lax.axis_index("c")
        base = wid * n_per_w
        pltpu.sync_copy(idx_hbm.at[wid], idx_v)
        for b in range(NBUF):
            pltpu.make_async_copy(
                table_hbm.at[idx_v.at[b]], rows[b], gsem[b]
            ).start()

        def ring_body(j, _):
            c = j * NBUF
            for b in range(NBUF):
                i = c + b
                pltpu.make_async_copy(
                    table_hbm.at[idx_v.at[0]], rows[b], gsem[b]
                ).wait()
                pltpu.make_async_copy(
                    rows[b], out_hbm.at[pl.ds(base + i * CHUNK, CHUNK)], ssem[b]
                ).start()

                @pl.when(i + NBUF < n_chunks)
                def _(b=b, i=i):
                    pltpu.make_async_copy(
                        rows[b],
                        out_hbm.at[pl.ds(base, CHUNK)],
                        ssem[b],
                    ).wait()
                    pltpu.make_async_copy(
                        table_hbm.at[idx_v.at[i + NBUF]], rows[b], gsem[b]
                    ).start()

            return 0

        lax.fori_loop(0, n_chunks // NBUF, ring_body, 0)
        for b in range(NBUF):
            pltpu.make_async_copy(
                rows[b], out_hbm.at[pl.ds(base, CHUNK)], ssem[b]
            ).wait()

    return emb


def kernel(token_ids, weights):
    orig_shape = token_ids.shape
    idx = token_ids.reshape(-1).astype(jnp.int32)
    B = idx.shape[0]
    idx3 = idx.reshape(NW, (B // NW) // CHUNK, CHUNK)
    out = _build(B)(weights, idx3)
    return out.reshape(*orig_shape, D)


# speedup vs baseline: 1.1223x; 1.1223x over previous
import functools

import jax
import jax.numpy as jnp
from jax import lax
from jax.experimental import pallas as pl
from jax.experimental.pallas import tpu as pltpu
from jax.experimental.pallas import tpu_sc as plsc

NUM_CORES = 2
NUM_SUBCORES = 16
NW = NUM_CORES * NUM_SUBCORES
CHUNK = 512
NBUF = 2

V = 1_000_000
D = 64
BLK = 1024
SPLIT = 512_000
N_LEFT = SPLIT // BLK
R_LAST = (V - SPLIT - 1) // BLK


def _pack_body(x1_ref, x2_ref, o_ref):
    o_ref[:, 0:D] = x1_ref[...].T
    o_ref[:, D : 2 * D] = x2_ref[...].T


def _pack(wt):
    return pl.pallas_call(
        _pack_body,
        grid=(N_LEFT,),
        in_specs=[
            pl.BlockSpec((D, BLK), lambda i: (0, i)),
            pl.BlockSpec((D, BLK), lambda i: (0, N_LEFT + jnp.minimum(i, R_LAST))),
        ],
        out_specs=pl.BlockSpec((BLK, 2 * D), lambda i: (i, 0)),
        out_shape=jax.ShapeDtypeStruct((SPLIT, 2 * D), jnp.float32),
    )(wt, wt)


@functools.lru_cache(maxsize=None)
def _gather(B):
    n_per_w = B // NW
    n_chunks = n_per_w // CHUNK
    assert n_chunks % NBUF == 0
    mesh = plsc.VectorSubcoreMesh(core_axis_name="c", subcore_axis_name="s")

    @functools.partial(
        pl.kernel,
        mesh=mesh,
        out_type=jax.ShapeDtypeStruct((B, D), jnp.float32),
        compiler_params=pltpu.CompilerParams(use_tc_tiling_on_sc=False),
        scratch_types=[
            pltpu.VMEM((n_chunks, CHUNK), jnp.int32),
        ]
        + [pltpu.VMEM((CHUNK, D), jnp.float32) for _ in range(NBUF)]
        + [pltpu.SemaphoreType.DMA for _ in range(2 * NBUF)],
    )
    def emb(table_hbm, idx_hbm, out_hbm, idx_v, *bufs_sems):
        rows = bufs_sems[:NBUF]
        gsem = bufs_sems[NBUF : 2 * NBUF]
        ssem = bufs_sems[2 * NBUF :]
        wid = lax.axis_index("s") * NUM_CORES + lax.axis_index("c")
        base = wid * n_per_w
        pltpu.sync_copy(idx_hbm.at[wid], idx_v)
        for b in range(NBUF):
            pltpu.make_async_copy(
                table_hbm.at[idx_v.at[b]], rows[b], gsem[b]
            ).start()

        def ring_body(j, _):
            c = j * NBUF
            for b in range(NBUF):
                i = c + b
                pltpu.make_async_copy(
                    table_hbm.at[idx_v.at[0]], rows[b], gsem[b]
                ).wait()
                pltpu.make_async_copy(
                    rows[b], out_hbm.at[pl.ds(base + i * CHUNK, CHUNK)], ssem[b]
                ).start()

                @pl.when(i + NBUF < n_chunks)
                def _(b=b, i=i):
                    pltpu.make_async_copy(
                        rows[b], out_hbm.at[pl.ds(base, CHUNK)], ssem[b]
                    ).wait()
                    pltpu.make_async_copy(
                        table_hbm.at[idx_v.at[i + NBUF]], rows[b], gsem[b]
                    ).start()

            return 0

        lax.fori_loop(0, n_chunks // NBUF, ring_body, 0)
        for b in range(NBUF):
            pltpu.make_async_copy(
                rows[b], out_hbm.at[pl.ds(base, CHUNK)], ssem[b]
            ).wait()

    return emb


def kernel(token_ids, weights):
    B, S = token_ids.shape
    packed = _pack(weights.T).reshape(2 * SPLIT, D)
    r = token_ids.astype(jnp.int32)
    r = jnp.where(r < SPLIT, 2 * r, 2 * (r - SPLIT) + 1)
    idx = r.reshape(NW, (B * S // NW) // CHUNK, CHUNK)
    out = _gather(B * S)(packed, idx)
    return out.reshape(B, S, D)


# speedup vs baseline: 1.2545x; 1.1178x over previous
import functools

import jax
import jax.numpy as jnp
from jax import lax
from jax.experimental import pallas as pl
from jax.experimental.pallas import tpu as pltpu
from jax.experimental.pallas import tpu_sc as plsc

NUM_CORES = 2
NUM_SUBCORES = 16
NW = NUM_CORES * NUM_SUBCORES
CHUNK = 512
NBUF = 2

V = 1_000_000
D = 64
BLK = 2048
SPLIT = 512_000
N_LEFT = SPLIT // BLK
R_LAST = (V - SPLIT - 1) // BLK


def _pack_body(x1_ref, x2_ref, o_ref):
    o_ref[:, 0:D] = x1_ref[...].T
    o_ref[:, D : 2 * D] = x2_ref[...].T


def _pack(wt):
    return pl.pallas_call(
        _pack_body,
        grid=(N_LEFT,),
        in_specs=[
            pl.BlockSpec((D, BLK), lambda i: (0, i)),
            pl.BlockSpec((D, BLK), lambda i: (0, N_LEFT + jnp.minimum(i, R_LAST))),
        ],
        out_specs=pl.BlockSpec((BLK, 2 * D), lambda i: (i, 0)),
        out_shape=jax.ShapeDtypeStruct((SPLIT, 2 * D), jnp.float32),
    )(wt, wt)


@functools.lru_cache(maxsize=None)
def _gather(B):
    n_per_w = B // NW
    n_chunks = n_per_w // CHUNK
    assert n_chunks % NBUF == 0
    mesh = plsc.VectorSubcoreMesh(core_axis_name="c", subcore_axis_name="s")

    @functools.partial(
        pl.kernel,
        mesh=mesh,
        out_type=jax.ShapeDtypeStruct((B, D), jnp.float32),
        compiler_params=pltpu.CompilerParams(use_tc_tiling_on_sc=False),
        scratch_types=[
            pltpu.VMEM((n_chunks, CHUNK), jnp.int32),
        ]
        + [pltpu.VMEM((CHUNK, D), jnp.float32) for _ in range(NBUF)]
        + [pltpu.SemaphoreType.DMA for _ in range(2 * NBUF)],
    )
    def emb(table_hbm, idx_hbm, out_hbm, idx_v, *bufs_sems):
        rows = bufs_sems[:NBUF]
        gsem = bufs_sems[NBUF : 2 * NBUF]
        ssem = bufs_sems[2 * NBUF :]
        wid = lax.axis_index("s") * NUM_CORES + lax.axis_index("c")
        base = wid * n_per_w
        pltpu.sync_copy(idx_hbm.at[wid], idx_v)
        for b in range(NBUF):
            pltpu.make_async_copy(
                table_hbm.at[idx_v.at[b]], rows[b], gsem[b]
            ).start()

        def ring_body(j, _):
            c = j * NBUF
            for b in range(NBUF):
                i = c + b
                pltpu.make_async_copy(
                    table_hbm.at[idx_v.at[0]], rows[b], gsem[b]
                ).wait()
                pltpu.make_async_copy(
                    rows[b], out_hbm.at[pl.ds(base + i * CHUNK, CHUNK)], ssem[b]
                ).start()

                @pl.when(i + NBUF < n_chunks)
                def _(b=b, i=i):
                    pltpu.make_async_copy(
                        rows[b], out_hbm.at[pl.ds(base, CHUNK)], ssem[b]
                    ).wait()
                    pltpu.make_async_copy(
                        table_hbm.at[idx_v.at[i + NBUF]], rows[b], gsem[b]
                    ).start()

            return 0

        lax.fori_loop(0, n_chunks // NBUF, ring_body, 0)
        for b in range(NBUF):
            pltpu.make_async_copy(
                rows[b], out_hbm.at[pl.ds(base, CHUNK)], ssem[b]
            ).wait()

    return emb


def kernel(token_ids, weights):
    B, S = token_ids.shape
    packed = _pack(weights.T).reshape(2 * SPLIT, D)
    r = token_ids.astype(jnp.int32)
    r = jnp.where(r < SPLIT, 2 * r, 2 * (r - SPLIT) + 1)
    idx = r.reshape(NW, (B * S // NW) // CHUNK, CHUNK)
    out = _gather(B * S)(packed, idx)
    return out.reshape(B, S, D)


# speedup vs baseline: 1.3469x; 1.0736x over previous
import functools

import jax
import jax.numpy as jnp
from jax import lax
from jax.experimental import pallas as pl
from jax.experimental.pallas import tpu as pltpu
from jax.experimental.pallas import tpu_sc as plsc

NUM_CORES = 2
NUM_SUBCORES = 16
NW = NUM_CORES * NUM_SUBCORES
CHUNK = 512
NBUF = 2

V = 1_000_000
D = 64
BLK = 4096
SPLIT = 512_000
N_LEFT = SPLIT // BLK
R_LAST = (V - SPLIT - 1) // BLK


def _pack_body(x1_ref, x2_ref, o_ref):
    o_ref[:, 0:D] = x1_ref[...].T
    o_ref[:, D : 2 * D] = x2_ref[...].T


def _pack(wt):
    return pl.pallas_call(
        _pack_body,
        grid=(N_LEFT,),
        in_specs=[
            pl.BlockSpec((D, BLK), lambda i: (0, i)),
            pl.BlockSpec((D, BLK), lambda i: (0, N_LEFT + jnp.minimum(i, R_LAST))),
        ],
        out_specs=pl.BlockSpec((BLK, 2 * D), lambda i: (i, 0)),
        out_shape=jax.ShapeDtypeStruct((SPLIT, 2 * D), jnp.float32),
    )(wt, wt)


@functools.lru_cache(maxsize=None)
def _gather(B):
    n_per_w = B // NW
    n_chunks = n_per_w // CHUNK
    assert n_chunks % NBUF == 0
    mesh = plsc.VectorSubcoreMesh(core_axis_name="c", subcore_axis_name="s")

    @functools.partial(
        pl.kernel,
        mesh=mesh,
        out_type=jax.ShapeDtypeStruct((B, D), jnp.float32),
        compiler_params=pltpu.CompilerParams(use_tc_tiling_on_sc=False),
        scratch_types=[
            pltpu.VMEM((n_chunks, CHUNK), jnp.int32),
        ]
        + [pltpu.VMEM((CHUNK, D), jnp.float32) for _ in range(NBUF)]
        + [pltpu.SemaphoreType.DMA for _ in range(2 * NBUF)],
    )
    def emb(table_hbm, idx_hbm, out_hbm, idx_v, *bufs_sems):
        rows = bufs_sems[:NBUF]
        gsem = bufs_sems[NBUF : 2 * NBUF]
        ssem = bufs_sems[2 * NBUF :]
        wid = lax.axis_index("s") * NUM_CORES + lax.axis_index("c")
        base = wid * n_per_w
        pltpu.sync_copy(idx_hbm.at[wid], idx_v)
        for b in range(NBUF):
            pltpu.make_async_copy(
                table_hbm.at[idx_v.at[b]], rows[b], gsem[b]
            ).start()

        def ring_body(j, _):
            c = j * NBUF
            for b in range(NBUF):
                i = c + b
                pltpu.make_async_copy(
                    table_hbm.at[idx_v.at[0]], rows[b], gsem[b]
                ).wait()
                pltpu.make_async_copy(
                    rows[b], out_hbm.at[pl.ds(base + i * CHUNK, CHUNK)], ssem[b]
                ).start()

                @pl.when(i + NBUF < n_chunks)
                def _(b=b, i=i):
                    pltpu.make_async_copy(
                        rows[b], out_hbm.at[pl.ds(base, CHUNK)], ssem[b]
                    ).wait()
                    pltpu.make_async_copy(
                        table_hbm.at[idx_v.at[i + NBUF]], rows[b], gsem[b]
                    ).start()

            return 0

        lax.fori_loop(0, n_chunks // NBUF, ring_body, 0)
        for b in range(NBUF):
            pltpu.make_async_copy(
                rows[b], out_hbm.at[pl.ds(base, CHUNK)], ssem[b]
            ).wait()

    return emb


def kernel(token_ids, weights):
    B, S = token_ids.shape
    packed = _pack(weights.T).reshape(2 * SPLIT, D)
    r = token_ids.astype(jnp.int32)
    r = jnp.where(r < SPLIT, 2 * r, 2 * (r - SPLIT) + 1)
    idx = r.reshape(NW, (B * S // NW) // CHUNK, CHUNK)
    out = _gather(B * S)(packed, idx)
    return out.reshape(B, S, D)


# speedup vs baseline: 1.3892x; 1.0314x over previous
import functools

import jax
import jax.numpy as jnp
from jax import lax
from jax.experimental import pallas as pl
from jax.experimental.pallas import tpu as pltpu
from jax.experimental.pallas import tpu_sc as plsc

NUM_CORES = 2
NUM_SUBCORES = 16
NW = NUM_CORES * NUM_SUBCORES
CHUNK = 512
NBUF = 2

V = 1_000_000
D = 64
BLK = 8192
SPLIT = 524_288
N_LEFT = SPLIT // BLK
R_LAST = (V - SPLIT - 1) // BLK


def _pack_body(x1_ref, x2_ref, o_ref):
    o_ref[:, 0:D] = x1_ref[...].T
    o_ref[:, D : 2 * D] = x2_ref[...].T


def _pack(wt):
    return pl.pallas_call(
        _pack_body,
        grid=(N_LEFT,),
        in_specs=[
            pl.BlockSpec((D, BLK), lambda i: (0, i)),
            pl.BlockSpec((D, BLK), lambda i: (0, N_LEFT + jnp.minimum(i, R_LAST))),
        ],
        out_specs=pl.BlockSpec((BLK, 2 * D), lambda i: (i, 0)),
        out_shape=jax.ShapeDtypeStruct((SPLIT, 2 * D), jnp.float32),
    )(wt, wt)


@functools.lru_cache(maxsize=None)
def _gather(B):
    n_per_w = B // NW
    n_chunks = n_per_w // CHUNK
    assert n_chunks % NBUF == 0
    mesh = plsc.VectorSubcoreMesh(core_axis_name="c", subcore_axis_name="s")

    @functools.partial(
        pl.kernel,
        mesh=mesh,
        out_type=jax.ShapeDtypeStruct((B, D), jnp.float32),
        compiler_params=pltpu.CompilerParams(use_tc_tiling_on_sc=False),
        scratch_types=[
            pltpu.VMEM((n_chunks, CHUNK), jnp.int32),
        ]
        + [pltpu.VMEM((CHUNK, D), jnp.float32) for _ in range(NBUF)]
        + [pltpu.SemaphoreType.DMA for _ in range(2 * NBUF)],
    )
    def emb(table_hbm, idx_hbm, out_hbm, idx_v, *bufs_sems):
        rows = bufs_sems[:NBUF]
        gsem = bufs_sems[NBUF : 2 * NBUF]
        ssem = bufs_sems[2 * NBUF :]
        wid = lax.axis_index("s") * NUM_CORES + lax.axis_index("c")
        base = wid * n_per_w
        pltpu.sync_copy(idx_hbm.at[wid], idx_v)
        for b in range(NBUF):
            pltpu.make_async_copy(
                table_hbm.at[idx_v.at[b]], rows[b], gsem[b]
            ).start()

        def ring_body(j, _):
            c = j * NBUF
            for b in range(NBUF):
                i = c + b
                pltpu.make_async_copy(
                    table_hbm.at[idx_v.at[0]], rows[b], gsem[b]
                ).wait()
                pltpu.make_async_copy(
                    rows[b], out_hbm.at[pl.ds(base + i * CHUNK, CHUNK)], ssem[b]
                ).start()

                @pl.when(i + NBUF < n_chunks)
                def _(b=b, i=i):
                    pltpu.make_async_copy(
                        rows[b], out_hbm.at[pl.ds(base, CHUNK)], ssem[b]
                    ).wait()
                    pltpu.make_async_copy(
                        table_hbm.at[idx_v.at[i + NBUF]], rows[b], gsem[b]
                    ).start()

            return 0

        lax.fori_loop(0, n_chunks // NBUF, ring_body, 0)
        for b in range(NBUF):
            pltpu.make_async_copy(
                rows[b], out_hbm.at[pl.ds(base, CHUNK)], ssem[b]
            ).wait()

    return emb


def kernel(token_ids, weights):
    B, S = token_ids.shape
    packed = _pack(weights.T).reshape(2 * SPLIT, D)
    r = token_ids.astype(jnp.int32)
    r = jnp.where(r < SPLIT, 2 * r, 2 * (r - SPLIT) + 1)
    idx = r.reshape(NW, (B * S // NW) // CHUNK, CHUNK)
    out = _gather(B * S)(packed, idx)
    return out.reshape(B, S, D)


# speedup vs baseline: 1.9998x; 1.4395x over previous
import functools

import jax
import jax.numpy as jnp
from jax import lax
from jax.experimental import pallas as pl
from jax.experimental.pallas import tpu as pltpu
from jax.experimental.pallas import tpu_sc as plsc

NUM_CORES = 2
NUM_SUBCORES = 16
NW = NUM_CORES * NUM_SUBCORES
CHUNK = 512
NBUF = 2

V = 1_000_000
D = 64
BLK = 8192
SPLIT = 524_288
N_LEFT = SPLIT // BLK
R_LAST = (V - SPLIT - 1) // BLK


def _pack_body(x1_ref, x2_ref, o_ref):
    o_ref[:, 0:D] = x1_ref[...].T
    o_ref[:, D : 2 * D] = x2_ref[...].T


def _pack(wt):
    return pl.pallas_call(
        _pack_body,
        grid=(N_LEFT,),
        in_specs=[
            pl.BlockSpec((D, BLK), lambda i: (0, i)),
            pl.BlockSpec((D, BLK), lambda i: (0, N_LEFT + jnp.minimum(i, R_LAST))),
        ],
        out_specs=pl.BlockSpec((BLK, 2 * D), lambda i: (i, 0)),
        out_shape=jax.ShapeDtypeStruct((SPLIT, 2 * D), jnp.float32),
    )(wt, wt)


@functools.lru_cache(maxsize=None)
def _gather(B):
    n_per_w = B // NW
    n_chunks = n_per_w // CHUNK
    assert n_chunks % NBUF == 0
    mesh = plsc.VectorSubcoreMesh(core_axis_name="c", subcore_axis_name="s")

    @functools.partial(
        pl.kernel,
        mesh=mesh,
        out_type=jax.ShapeDtypeStruct((B, D), jnp.float32),
        compiler_params=pltpu.CompilerParams(use_tc_tiling_on_sc=False),
        scratch_types=[
            pltpu.VMEM((n_chunks, CHUNK), jnp.int32),
        ]
        + [pltpu.VMEM((CHUNK, D), jnp.float32) for _ in range(NBUF)]
        + [pltpu.SemaphoreType.DMA for _ in range(2 * NBUF)],
    )
    def emb(table_hbm, idx_hbm, out_hbm, idx_v, *bufs_sems):
        rows = bufs_sems[:NBUF]
        gsem = bufs_sems[NBUF : 2 * NBUF]
        ssem = bufs_sems[2 * NBUF :]
        wid = lax.axis_index("s") * NUM_CORES + lax.axis_index("c")
        base = wid * n_per_w
        pltpu.sync_copy(idx_hbm.at[wid], idx_v)
        for b in range(NBUF):
            pltpu.make_async_copy(
                table_hbm.at[idx_v.at[b]], rows[b], gsem[b]
            ).start()

        def ring_body(j, _):
            c = j * NBUF
            for b in range(NBUF):
                i = c + b
                pltpu.make_async_copy(
                    table_hbm.at[idx_v.at[0]], rows[b], gsem[b]
                ).wait()
                pltpu.make_async_copy(
                    rows[b], out_hbm.at[pl.ds(base + i * CHUNK, CHUNK)], ssem[b]
                ).start()

                @pl.when(i + NBUF < n_chunks)
                def _(b=b, i=i):
                    pltpu.make_async_copy(
                        rows[b], out_hbm.at[pl.ds(base, CHUNK)], ssem[b]
                    ).wait()
                    pltpu.make_async_copy(
                        table_hbm.at[idx_v.at[i + NBUF]], rows[b], gsem[b]
                    ).start()

            return 0

        lax.fori_loop(0, n_chunks // NBUF, ring_body, 0)
        for b in range(NBUF):
            pltpu.make_async_copy(
                rows[b], out_hbm.at[pl.ds(base, CHUNK)], ssem[b]
            ).wait()

    return emb


BQ = 128


def _unpack_body(x_ref, o_ref):
    S = 50
    for s in range(S):
        blk = x_ref[pl.Slice(s // 2, BQ, S // 2), :]
        half = jax.lax.slice(blk, (0, (s % 2) * D), (BQ, (s % 2 + 1) * D))
        o_ref[s, :, :] = half.T


def _unpack(x, B, S):
    return pl.pallas_call(
        _unpack_body,
        grid=(B // BQ,),
        in_specs=[pl.BlockSpec((BQ * S // 2, 2 * D), lambda i: (i, 0))],
        out_specs=pl.BlockSpec((S, D, BQ), lambda i: (0, 0, i)),
        out_shape=jax.ShapeDtypeStruct((S, D, B), jnp.float32),
    )(x)


def kernel(token_ids, weights):
    B, S = token_ids.shape
    packed = _pack(weights.T).reshape(2 * SPLIT, D)
    r = token_ids.astype(jnp.int32)
    r = jnp.where(r < SPLIT, 2 * r, 2 * (r - SPLIT) + 1)
    idx = r.reshape(NW, (B * S // NW) // CHUNK, CHUNK)
    out = _gather(B * S)(packed, idx)
    out_t = _unpack(out.reshape(B * S // 2, 2 * D), B, S)
    return out_t.transpose(2, 0, 1)


# speedup vs baseline: 2.1185x; 1.0593x over previous
import functools

import jax
import jax.numpy as jnp
from jax import lax
from jax.experimental import pallas as pl
from jax.experimental.pallas import tpu as pltpu
from jax.experimental.pallas import tpu_sc as plsc

NUM_CORES = 2
NUM_SUBCORES = 16
NW = NUM_CORES * NUM_SUBCORES
CHUNK = 512
NBUF = 2

V = 1_000_000
D = 64
BLK = 8192
SPLIT = 524_288
N_LEFT = SPLIT // BLK
R_LAST = (V - SPLIT - 1) // BLK


def _pack_body(x1_ref, x2_ref, o_ref):
    o_ref[:, 0:D] = x1_ref[...].T
    o_ref[:, D : 2 * D] = x2_ref[...].T


def _pack(wt):
    return pl.pallas_call(
        _pack_body,
        grid=(N_LEFT,),
        in_specs=[
            pl.BlockSpec((D, BLK), lambda i: (0, i)),
            pl.BlockSpec((D, BLK), lambda i: (0, N_LEFT + jnp.minimum(i, R_LAST))),
        ],
        out_specs=pl.BlockSpec((BLK, 2 * D), lambda i: (i, 0)),
        out_shape=jax.ShapeDtypeStruct((SPLIT, 2 * D), jnp.float32),
    )(wt, wt)


@functools.lru_cache(maxsize=None)
def _gather(B):
    n_per_w = B // NW
    n_chunks = n_per_w // CHUNK
    assert n_chunks % NBUF == 0
    mesh = plsc.VectorSubcoreMesh(core_axis_name="c", subcore_axis_name="s")

    @functools.partial(
        pl.kernel,
        mesh=mesh,
        out_type=jax.ShapeDtypeStruct((B, D), jnp.float32),
        compiler_params=pltpu.CompilerParams(use_tc_tiling_on_sc=False),
        scratch_types=[
            pltpu.VMEM((n_chunks, CHUNK), jnp.int32),
        ]
        + [pltpu.VMEM((CHUNK, D), jnp.float32) for _ in range(NBUF)]
        + [pltpu.SemaphoreType.DMA for _ in range(2 * NBUF)],
    )
    def emb(table_hbm, idx_hbm, out_hbm, idx_v, *bufs_sems):
        rows = bufs_sems[:NBUF]
        gsem = bufs_sems[NBUF : 2 * NBUF]
        ssem = bufs_sems[2 * NBUF :]
        wid = lax.axis_index("s") * NUM_CORES + lax.axis_index("c")
        base = wid * n_per_w
        pltpu.sync_copy(idx_hbm.at[wid], idx_v)
        for b in range(NBUF):
            pltpu.make_async_copy(
                table_hbm.at[idx_v.at[b]], rows[b], gsem[b]
            ).start()

        def ring_body(j, _):
            c = j * NBUF
            for b in range(NBUF):
                i = c + b
                pltpu.make_async_copy(
                    table_hbm.at[idx_v.at[0]], rows[b], gsem[b]
                ).wait()
                pltpu.make_async_copy(
                    rows[b], out_hbm.at[pl.ds(base + i * CHUNK, CHUNK)], ssem[b]
                ).start()

                @pl.when(i + NBUF < n_chunks)
                def _(b=b, i=i):
                    pltpu.make_async_copy(
                        rows[b], out_hbm.at[pl.ds(base, CHUNK)], ssem[b]
                    ).wait()
                    pltpu.make_async_copy(
                        table_hbm.at[idx_v.at[i + NBUF]], rows[b], gsem[b]
                    ).start()

            return 0

        lax.fori_loop(0, n_chunks // NBUF, ring_body, 0)
        for b in range(NBUF):
            pltpu.make_async_copy(
                rows[b], out_hbm.at[pl.ds(base, CHUNK)], ssem[b]
            ).wait()

    return emb


BQ = 256


def _unpack_body(x_ref, o_ref):
    S = 50
    for s in range(S):
        blk = x_ref[pl.Slice(s // 2, BQ, S // 2), :]
        half = jax.lax.slice(blk, (0, (s % 2) * D), (BQ, (s % 2 + 1) * D))
        o_ref[s, :, :] = half.T


def _unpack(x, B, S):
    return pl.pallas_call(
        _unpack_body,
        grid=(B // BQ,),
        in_specs=[pl.BlockSpec((BQ * S // 2, 2 * D), lambda i: (i, 0))],
        out_specs=pl.BlockSpec((S, D, BQ), lambda i: (0, 0, i)),
        out_shape=jax.ShapeDtypeStruct((S, D, B), jnp.float32),
    )(x)


def kernel(token_ids, weights):
    B, S = token_ids.shape
    packed = _pack(weights.T).reshape(2 * SPLIT, D)
    r = token_ids.astype(jnp.int32)
    r = jnp.where(r < SPLIT, 2 * r, 2 * (r - SPLIT) + 1)
    idx = r.reshape(NW, (B * S // NW) // CHUNK, CHUNK)
    out = _gather(B * S)(packed, idx)
    out_t = _unpack(out.reshape(B * S // 2, 2 * D), B, S)
    return out_t.transpose(2, 0, 1)


# speedup vs baseline: 2.1799x; 1.0290x over previous
import functools

import jax
import jax.numpy as jnp
from jax import lax
from jax.experimental import pallas as pl
from jax.experimental.pallas import tpu as pltpu
from jax.experimental.pallas import tpu_sc as plsc

NUM_CORES = 2
NUM_SUBCORES = 16
NW = NUM_CORES * NUM_SUBCORES
CHUNK = 512
NBUF = 2

V = 1_000_000
D = 64
BLK = 8192
SPLIT = 524_288
N_LEFT = SPLIT // BLK
R_LAST = (V - SPLIT - 1) // BLK


def _pack_body(x1_ref, x2_ref, o_ref):
    o_ref[:, 0:D] = x1_ref[...].T
    o_ref[:, D : 2 * D] = x2_ref[...].T


def _pack(wt):
    return pl.pallas_call(
        _pack_body,
        grid=(N_LEFT,),
        in_specs=[
            pl.BlockSpec((D, BLK), lambda i: (0, i)),
            pl.BlockSpec((D, BLK), lambda i: (0, N_LEFT + jnp.minimum(i, R_LAST))),
        ],
        out_specs=pl.BlockSpec((BLK, 2 * D), lambda i: (i, 0)),
        out_shape=jax.ShapeDtypeStruct((SPLIT, 2 * D), jnp.float32),
    )(wt, wt)


@functools.lru_cache(maxsize=None)
def _gather(B):
    n_per_w = B // NW
    n_chunks = n_per_w // CHUNK
    assert n_chunks % NBUF == 0
    mesh = plsc.VectorSubcoreMesh(core_axis_name="c", subcore_axis_name="s")

    @functools.partial(
        pl.kernel,
        mesh=mesh,
        out_type=jax.ShapeDtypeStruct((B, D), jnp.float32),
        compiler_params=pltpu.CompilerParams(use_tc_tiling_on_sc=False),
        scratch_types=[
            pltpu.VMEM((n_chunks, CHUNK), jnp.int32),
        ]
        + [pltpu.VMEM((CHUNK, D), jnp.float32) for _ in range(NBUF)]
        + [pltpu.SemaphoreType.DMA for _ in range(2 * NBUF)],
    )
    def emb(table_hbm, idx_hbm, out_hbm, idx_v, *bufs_sems):
        rows = bufs_sems[:NBUF]
        gsem = bufs_sems[NBUF : 2 * NBUF]
        ssem = bufs_sems[2 * NBUF :]
        wid = lax.axis_index("s") * NUM_CORES + lax.axis_index("c")
        base = wid * n_per_w
        pltpu.sync_copy(idx_hbm.at[wid], idx_v)
        for b in range(NBUF):
            pltpu.make_async_copy(
                table_hbm.at[idx_v.at[b]], rows[b], gsem[b]
            ).start()

        def ring_body(j, _):
            c = j * NBUF
            for b in range(NBUF):
                i = c + b
                pltpu.make_async_copy(
                    table_hbm.at[idx_v.at[0]], rows[b], gsem[b]
                ).wait()
                pltpu.make_async_copy(
                    rows[b], out_hbm.at[pl.ds(base + i * CHUNK, CHUNK)], ssem[b]
                ).start()

                @pl.when(i + NBUF < n_chunks)
                def _(b=b, i=i):
                    pltpu.make_async_copy(
                        rows[b], out_hbm.at[pl.ds(base, CHUNK)], ssem[b]
                    ).wait()
                    pltpu.make_async_copy(
                        table_hbm.at[idx_v.at[i + NBUF]], rows[b], gsem[b]
                    ).start()

            return 0

        lax.fori_loop(0, n_chunks // NBUF, ring_body, 0)
        for b in range(NBUF):
            pltpu.make_async_copy(
                rows[b], out_hbm.at[pl.ds(base, CHUNK)], ssem[b]
            ).wait()

    return emb


BQ = 512


def _unpack_body(x_ref, o_ref):
    S = 50
    for s in range(S):
        blk = x_ref[pl.Slice(s // 2, BQ, S // 2), :]
        half = jax.lax.slice(blk, (0, (s % 2) * D), (BQ, (s % 2 + 1) * D))
        o_ref[s, :, :] = half.T


def _unpack(x, B, S):
    return pl.pallas_call(
        _unpack_body,
        grid=(B // BQ,),
        in_specs=[pl.BlockSpec((BQ * S // 2, 2 * D), lambda i: (i, 0))],
        out_specs=pl.BlockSpec((S, D, BQ), lambda i: (0, 0, i)),
        out_shape=jax.ShapeDtypeStruct((S, D, B), jnp.float32),
    )(x)


def kernel(token_ids, weights):
    B, S = token_ids.shape
    packed = _pack(weights.T).reshape(2 * SPLIT, D)
    r = token_ids.astype(jnp.int32)
    r = jnp.where(r < SPLIT, 2 * r, 2 * (r - SPLIT) + 1)
    idx = r.reshape(NW, (B * S // NW) // CHUNK, CHUNK)
    out = _gather(B * S)(packed, idx)
    out_t = _unpack(out.reshape(B * S // 2, 2 * D), B, S)
    return out_t.transpose(2, 0, 1)
